# Initial kernel scaffold; baseline (speedup 1.0000x reference)
import jax
import jax.numpy as jnp
from jax.experimental import pallas as pl


def kernel(z, W):
    raise NotImplementedError("write your pallas kernel here")


# speedup vs baseline: 1.2218x; 1.2218x over previous
import jax
import jax.numpy as jnp
from jax.experimental import pallas as pl


def _vq_body(z_ref, w_ref, q_ref, idx_ref, loss_ref):
    z = z_ref[0]
    w = w_ref[...]
    ncodes = w.shape[0]

    wsq = jnp.sum(w * w, axis=1, keepdims=True)
    zsq = jnp.sum(z * z, axis=0, keepdims=True)
    mm = jax.lax.dot_general(
        w, z, (((1,), (0,)), ((), ())),
        preferred_element_type=jnp.float32)
    d = (zsq + wsq) - 2.0 * mm

    m = jnp.min(d, axis=0, keepdims=True)
    iota = jax.lax.broadcasted_iota(jnp.int32, d.shape, 0)
    cand = jnp.where(d == m, iota, ncodes)
    idx = jnp.min(cand, axis=0)

    oh = (iota == idx[None, :]).astype(jnp.float32)
    q = jax.lax.dot_general(
        w, oh, (((0,), (0,)), ((), ())),
        preferred_element_type=jnp.float32)

    diff = q - z
    q_ref[0] = z + diff
    idx_ref[0, 0] = idx
    loss_ref[...] = jnp.sum(diff * diff).reshape(1, 1, 1)


def kernel(z, W):
    B, C, H, Wsp = z.shape
    HW = H * Wsp
    ncodes = W.shape[0]
    zr = z.reshape(B, C, HW)

    q, idx, losses = pl.pallas_call(
        _vq_body,
        grid=(B,),
        in_specs=[
            pl.BlockSpec((1, C, HW), lambda b: (b, 0, 0)),
            pl.BlockSpec((ncodes, C), lambda b: (0, 0)),
        ],
        out_specs=[
            pl.BlockSpec((1, C, HW), lambda b: (b, 0, 0)),
            pl.BlockSpec((1, 1, HW), lambda b: (b, 0, 0)),
            pl.BlockSpec((1, 1, 1), lambda b: (b, 0, 0)),
        ],
        out_shape=[
            jax.ShapeDtypeStruct((B, C, HW), jnp.float32),
            jax.ShapeDtypeStruct((B, 1, HW), jnp.int32),
            jax.ShapeDtypeStruct((B, 1, 1), jnp.float32),
        ],
    )(zr, W)

    q_out = q.reshape(B, C, H, Wsp)
    idx_out = idx.reshape(B, H, Wsp)
    loss = jnp.sum(losses) / (B * C * HW)
    return (q_out, loss, loss, idx_out)


# speedup vs baseline: 1.2226x; 1.0006x over previous
import jax
import jax.numpy as jnp
from jax.experimental import pallas as pl
from jax.experimental.pallas import tpu as pltpu


def _vq_body(z_ref, w_ref, q_ref, idx_ref, loss_ref):
    z = z_ref[0]
    w = w_ref[...]
    ncodes = w.shape[0]

    wsq_h = jnp.sum(w * w, axis=1, keepdims=True) * 0.5
    zsq_h = jnp.sum(z * z, axis=0, keepdims=True) * 0.5
    mm = jax.lax.dot_general(
        w, z, (((1,), (0,)), ((), ())),
        preferred_element_type=jnp.float32)
    d = (zsq_h + wsq_h) - mm

    m = jnp.min(d, axis=0, keepdims=True)
    iota = jax.lax.broadcasted_iota(jnp.int32, d.shape, 0)
    cand = jnp.where(d == m, iota, ncodes)
    idx = jnp.min(cand, axis=0)

    oh = (iota == idx[None, :]).astype(jnp.bfloat16)
    q = jax.lax.dot_general(
        w.astype(jnp.bfloat16), oh, (((0,), (0,)), ((), ())),
        preferred_element_type=jnp.float32)

    diff = q - z
    q_ref[0] = z + diff
    idx_ref[0, 0] = idx
    loss_ref[...] = jnp.sum(diff * diff).reshape(1, 1, 1)


def kernel(z, W):
    B, C, H, Wsp = z.shape
    HW = H * Wsp
    ncodes = W.shape[0]
    zr = z.reshape(B, C, HW)

    q, idx, losses = pl.pallas_call(
        _vq_body,
        grid=(B,),
        in_specs=[
            pl.BlockSpec((1, C, HW), lambda b: (b, 0, 0)),
            pl.BlockSpec((ncodes, C), lambda b: (0, 0)),
        ],
        out_specs=[
            pl.BlockSpec((1, C, HW), lambda b: (b, 0, 0)),
            pl.BlockSpec((1, 1, HW), lambda b: (b, 0, 0)),
            pl.BlockSpec((1, 1, 1), lambda b: (b, 0, 0)),
        ],
        out_shape=[
            jax.ShapeDtypeStruct((B, C, HW), jnp.float32),
            jax.ShapeDtypeStruct((B, 1, HW), jnp.int32),
            jax.ShapeDtypeStruct((B, 1, 1), jnp.float32),
        ],
        compiler_params=pltpu.CompilerParams(
            dimension_semantics=("parallel",),
        ),
    )(zr, W)

    q_out = q.reshape(B, C, H, Wsp)
    idx_out = idx.reshape(B, H, Wsp)
    loss = jnp.sum(losses) / (B * C * HW)
    return (q_out, loss, loss, idx_out)


# speedup vs baseline: 1.2732x; 1.0413x over previous
import jax
import jax.numpy as jnp
from jax.experimental import pallas as pl
from jax.experimental.pallas import tpu as pltpu


def _vq_body(z_ref, w_ref, q_ref, idx_ref, loss_ref):
    nb = z_ref.shape[0]
    w = w_ref[...]
    ncodes = w.shape[0]

    wsq_h = jnp.sum(w * w, axis=1, keepdims=True) * 0.5
    w_bf = w.astype(jnp.bfloat16)
    loss_acc = jnp.float32(0.0)

    for b in range(nb):
        z = z_ref[b]
        zsq_h = jnp.sum(z * z, axis=0, keepdims=True) * 0.5
        mm = jax.lax.dot_general(
            w, z, (((1,), (0,)), ((), ())),
            preferred_element_type=jnp.float32)
        d = (zsq_h + wsq_h) - mm

        m = jnp.min(d, axis=0, keepdims=True)
        iota = jax.lax.broadcasted_iota(jnp.int32, d.shape, 0)
        cand = jnp.where(d == m, iota, ncodes)
        idx = jnp.min(cand, axis=0)

        oh = (iota == idx[None, :]).astype(jnp.bfloat16)
        q = jax.lax.dot_general(
            w_bf, oh, (((0,), (0,)), ((), ())),
            preferred_element_type=jnp.float32)

        diff = q - z
        q_ref[b] = z + diff
        idx_ref[b, 0] = idx
        loss_acc = loss_acc + jnp.sum(diff * diff)

    loss_ref[...] = loss_acc.reshape(1, 1, 1)


def kernel(z, W):
    B, C, H, Wsp = z.shape
    HW = H * Wsp
    ncodes = W.shape[0]
    zr = z.reshape(B, C, HW)

    NB = 4
    q, idx, losses = pl.pallas_call(
        _vq_body,
        grid=(B // NB,),
        in_specs=[
            pl.BlockSpec((NB, C, HW), lambda b: (b, 0, 0)),
            pl.BlockSpec((ncodes, C), lambda b: (0, 0)),
        ],
        out_specs=[
            pl.BlockSpec((NB, C, HW), lambda b: (b, 0, 0)),
            pl.BlockSpec((NB, 1, HW), lambda b: (b, 0, 0)),
            pl.BlockSpec((1, 1, 1), lambda b: (b, 0, 0)),
        ],
        out_shape=[
            jax.ShapeDtypeStruct((B, C, HW), jnp.float32),
            jax.ShapeDtypeStruct((B, 1, HW), jnp.int32),
            jax.ShapeDtypeStruct((B // NB, 1, 1), jnp.float32),
        ],
        compiler_params=pltpu.CompilerParams(
            dimension_semantics=("parallel",),
        ),
    )(zr, W)

    q_out = q.reshape(B, C, H, Wsp)
    idx_out = idx.reshape(B, H, Wsp)
    loss = jnp.sum(losses) / (B * C * HW)
    return (q_out, loss, loss, idx_out)
